# baseline (speedup 1.0000x reference)
import jax
import jax.numpy as jnp
from jax import lax
from jax.experimental import pallas as pl
from jax.experimental.pallas import tpu as pltpu
from jax.experimental.pallas import tpu_sc as plsc

_GAMMA = 12.0
_HIDDEN = 128
_BATCH = 16384
_NC = 2
_NS = 16
_NW = _NC * _NS
_ROWS_PER_W = _BATCH // _NW
_CHUNK = 128
_NCHUNK = _ROWS_PER_W // _CHUNK
_UNROLL = 8


def _sc_body(head_hbm, rel_hbm, tail_hbm, ent_hbm, reltab_hbm, out_hbm,
             idx_h, idx_r, idx_t, h_buf, r_buf, t_buf, out_v,
             sem0, sem1, sem2):
    wid = lax.axis_index("s") * _NC + lax.axis_index("c")
    wbase = pl.multiple_of(wid * _ROWS_PER_W, _ROWS_PER_W)
    lane = lax.iota(jnp.int32, 16)

    def chunk_body(c, _):
        base = pl.multiple_of(wbase + c * _CHUNK, _CHUNK)
        pltpu.sync_copy(head_hbm.at[pl.ds(base, _CHUNK)], idx_h)
        pltpu.sync_copy(rel_hbm.at[pl.ds(base, _CHUNK)], idx_r)
        pltpu.sync_copy(tail_hbm.at[pl.ds(base, _CHUNK)], idx_t)
        cp0 = pltpu.async_copy(ent_hbm.at[idx_h], h_buf, sem0)
        cp1 = pltpu.async_copy(reltab_hbm.at[idx_r], r_buf, sem1)
        cp2 = pltpu.async_copy(ent_hbm.at[idx_t], t_buf, sem2)
        cp0.wait()
        cp1.wait()
        cp2.wait()

        def group_body(g, _):
            rows = g * 16 + lane

            def j_body(jb, acc):
                for jj in range(_UNROLL):
                    col = jnp.full((16,), jb * _UNROLL + jj, jnp.int32)
                    hv = plsc.load_gather(h_buf, [rows, col])
                    rv = plsc.load_gather(r_buf, [rows, col])
                    tv = plsc.load_gather(t_buf, [rows, col])
                    acc = acc + jnp.abs(hv + rv - tv)
                return acc

            acc = lax.fori_loop(0, _HIDDEN // _UNROLL, j_body,
                                jnp.zeros((16,), jnp.float32))
            off = pl.multiple_of(c * _CHUNK + g * 16, 16)
            out_v[pl.ds(off, 16)] = _GAMMA - acc
            return 0

        lax.fori_loop(0, _CHUNK // 16, group_body, 0)
        return 0

    lax.fori_loop(0, _NCHUNK, chunk_body, 0)
    pltpu.sync_copy(out_v, out_hbm.at[pl.ds(wbase, _ROWS_PER_W)])


@jax.jit
def _run(head_idx, rel_idx, tail_idx, entity_embedding, relation_embedding):
    mesh = plsc.VectorSubcoreMesh(core_axis_name="c", subcore_axis_name="s")
    f = pl.kernel(
        _sc_body,
        out_type=jax.ShapeDtypeStruct((_BATCH,), jnp.float32),
        mesh=mesh,
        compiler_params=pltpu.CompilerParams(needs_layout_passes=False),
        scratch_types=[
            pltpu.VMEM((_CHUNK,), jnp.int32),
            pltpu.VMEM((_CHUNK,), jnp.int32),
            pltpu.VMEM((_CHUNK,), jnp.int32),
            pltpu.VMEM((_CHUNK, _HIDDEN), jnp.float32),
            pltpu.VMEM((_CHUNK, _HIDDEN), jnp.float32),
            pltpu.VMEM((_CHUNK, _HIDDEN), jnp.float32),
            pltpu.VMEM((_ROWS_PER_W,), jnp.float32),
            pltpu.SemaphoreType.DMA,
            pltpu.SemaphoreType.DMA,
            pltpu.SemaphoreType.DMA,
        ],
    )
    return f(head_idx, rel_idx, tail_idx, entity_embedding, relation_embedding)


def kernel(sample, entity_embedding, relation_embedding):
    head_idx = sample[:, 0]
    rel_idx = sample[:, 1]
    tail_idx = sample[:, 2]
    out = _run(head_idx, rel_idx, tail_idx, entity_embedding,
               relation_embedding)
    return out[:, None]


# speedup vs baseline: 3.3493x; 3.3493x over previous
import jax
import jax.numpy as jnp
from jax import lax
from jax.experimental import pallas as pl
from jax.experimental.pallas import tpu as pltpu
from jax.experimental.pallas import tpu_sc as plsc

_GAMMA = 12.0
_HIDDEN = 128
_BATCH = 16384
_NC = 2
_NS = 16
_NW = _NC * _NS
_ROWS_PER_W = _BATCH // _NW
_CHUNK = 128
_NCHUNK = _ROWS_PER_W // _CHUNK
_ROW_UNROLL = 4


def _sc_body(head_hbm, rel_hbm, tail_hbm, ent_hbm, reltab_hbm, out_hbm,
             idx_h, idx_r, idx_t, h_bufs, r_bufs, t_bufs, out_v, sems):
    wid = lax.axis_index("s") * _NC + lax.axis_index("c")
    wbase = pl.multiple_of(wid * _ROWS_PER_W, _ROWS_PER_W)
    lane = lax.iota(jnp.int32, 16)

    pltpu.sync_copy(head_hbm.at[pl.ds(wbase, _ROWS_PER_W)], idx_h)
    pltpu.sync_copy(rel_hbm.at[pl.ds(wbase, _ROWS_PER_W)], idx_r)
    pltpu.sync_copy(tail_hbm.at[pl.ds(wbase, _ROWS_PER_W)], idx_t)

    def launch(c):
        p = c % 2
        sl = pl.ds(c * _CHUNK, _CHUNK)
        return (
            pltpu.async_copy(ent_hbm.at[idx_h.at[sl]], h_bufs[p], sems[3 * p]),
            pltpu.async_copy(reltab_hbm.at[idx_r.at[sl]], r_bufs[p],
                             sems[3 * p + 1]),
            pltpu.async_copy(ent_hbm.at[idx_t.at[sl]], t_bufs[p],
                             sems[3 * p + 2]),
        )

    inflight = {0: launch(0)}
    if _NCHUNK > 1:
        inflight[1] = launch(1)

    for c in range(_NCHUNK):
        p = c % 2
        for cp in inflight.pop(c):
            cp.wait()
        h_buf, r_buf, t_buf = h_bufs[p], r_bufs[p], t_bufs[p]

        def group_body(g, _, c=c, h_buf=h_buf, r_buf=r_buf, t_buf=t_buf):
            def row_body(q, v):
                for u in range(_ROW_UNROLL):
                    rr = q * _ROW_UNROLL + u
                    row = g * 16 + rr
                    acc = jnp.zeros((16,), jnp.float32)
                    for k in range(_HIDDEN // 16):
                        sl = pl.ds(k * 16, 16)
                        acc = acc + jnp.abs(
                            h_buf[row, sl] + r_buf[row, sl] - t_buf[row, sl])
                    s = _GAMMA - jnp.sum(acc)
                    v = jnp.where(lane == rr, s, v)
                return v

            v = lax.fori_loop(0, 16 // _ROW_UNROLL, row_body,
                              jnp.zeros((16,), jnp.float32))
            off = pl.multiple_of(c * _CHUNK + g * 16, 16)
            out_v[pl.ds(off, 16)] = v
            return 0

        lax.fori_loop(0, _CHUNK // 16, group_body, 0)
        if c + 2 < _NCHUNK:
            inflight[c + 2] = launch(c + 2)

    pltpu.sync_copy(out_v, out_hbm.at[pl.ds(wbase, _ROWS_PER_W)])


@jax.jit
def _run(head_idx, rel_idx, tail_idx, entity_embedding, relation_embedding):
    mesh = plsc.VectorSubcoreMesh(core_axis_name="c", subcore_axis_name="s")
    f = pl.kernel(
        _sc_body,
        out_type=jax.ShapeDtypeStruct((_BATCH,), jnp.float32),
        mesh=mesh,
        compiler_params=pltpu.CompilerParams(needs_layout_passes=False),
        scratch_types=[
            pltpu.VMEM((_ROWS_PER_W,), jnp.int32),
            pltpu.VMEM((_ROWS_PER_W,), jnp.int32),
            pltpu.VMEM((_ROWS_PER_W,), jnp.int32),
            [pltpu.VMEM((_CHUNK, _HIDDEN), jnp.float32) for _ in range(2)],
            [pltpu.VMEM((_CHUNK, _HIDDEN), jnp.float32) for _ in range(2)],
            [pltpu.VMEM((_CHUNK, _HIDDEN), jnp.float32) for _ in range(2)],
            pltpu.VMEM((_ROWS_PER_W,), jnp.float32),
            [pltpu.SemaphoreType.DMA for _ in range(6)],
        ],
    )
    return f(head_idx, rel_idx, tail_idx, entity_embedding, relation_embedding)


def kernel(sample, entity_embedding, relation_embedding):
    head_idx = sample[:, 0]
    rel_idx = sample[:, 1]
    tail_idx = sample[:, 2]
    out = _run(head_idx, rel_idx, tail_idx, entity_embedding,
               relation_embedding)
    return out[:, None]
